# baseline (speedup 1.0000x reference)
import functools

import jax
import jax.numpy as jnp
from jax import lax
from jax.experimental import pallas as pl
from jax.experimental.pallas import tpu as pltpu
from jax.experimental.pallas import tpu_sc as plsc

_N = 10000
_H = 128
_G = 64
_OUT = 64
_NL = 3

_NC = 2
_NS = 16
_NW = _NC * _NS
_BLK = 128
_NPAD = _N + 112
_RPT = _NPAD // _NS

_RB = 1000
_GRID = _N // _RB

_MM = (((1,), (0,)), ((), ()))
_PREC = lax.Precision.HIGHEST


_NBUF = 2
_ISLOT = 2 * _NBUF
_NBLK_PAD = 80


def _make_agg(nblk):
    mesh = plsc.VectorSubcoreMesh(core_axis_name="c", subcore_axis_name="s",
                                  num_cores=_NC, num_subcores=_NS)

    @functools.partial(
        pl.kernel,
        mesh=mesh,
        out_type=jax.ShapeDtypeStruct((_NC, _NPAD, _H), jnp.float32),
        scratch_types=[
            pltpu.VMEM((_ISLOT, _BLK), jnp.int32),
            pltpu.VMEM((_ISLOT, _BLK), jnp.int32),
            pltpu.VMEM((_NBUF, _BLK, _H), jnp.float32),
            pltpu.VMEM_SHARED((_NPAD, _H), jnp.float32),
        ] + [pltpu.SemaphoreType.DMA] * (_NBUF + _ISLOT),
    )
    def agg(h_hbm, src_hbm, dst_hbm, zeros_hbm, out_hbm,
            src_r, dst_r, rows_v, acc_sh, *sems):
        gsems = sems[:_NBUF]
        isems = sems[_NBUF:]
        c = lax.axis_index("c")
        s = lax.axis_index("s")
        wid = c * _NS + s
        pltpu.sync_copy(zeros_hbm.at[pl.ds(s * _RPT, _RPT)],
                        acc_sh.at[pl.ds(s * _RPT, _RPT)])
        plsc.subcore_barrier()

        def fetch_idx(j, slot):
            pltpu.async_copy(src_hbm.at[wid].at[j], src_r.at[slot],
                             isems[slot])
            pltpu.async_copy(dst_hbm.at[wid].at[j], dst_r.at[slot],
                             isems[slot])

        def wait_idx(j, slot):
            pltpu.make_async_copy(src_hbm.at[wid].at[j], src_r.at[slot],
                                  isems[slot]).wait()
            pltpu.make_async_copy(dst_hbm.at[wid].at[j], dst_r.at[slot],
                                  isems[slot]).wait()

        def fetch_rows(slot, b):
            pltpu.async_copy(h_hbm.at[src_r.at[slot]], rows_v.at[b], gsems[b])

        for t in range(_ISLOT):
            fetch_idx(t, t)
        for b in range(_NBUF):
            wait_idx(b, b)
            fetch_rows(b, b)

        def round_body(r, carry):
            j0 = r * _ISLOT
            for t in range(_ISLOT):
                j = j0 + t
                slot = t
                b = t % _NBUF
                pltpu.make_async_copy(h_hbm.at[src_r.at[slot]], rows_v.at[b],
                                      gsems[b]).wait()
                pltpu.sync_copy(rows_v.at[b], acc_sh.at[dst_r.at[slot]],
                                add=True)

                @pl.when(j + _ISLOT < nblk)
                def _():
                    fetch_idx(j + _ISLOT, slot)

                jn = j + _NBUF
                sn = (t + _NBUF) % _ISLOT

                @pl.when(jn < nblk)
                def _():
                    wait_idx(jn, sn)
                    fetch_rows(sn, b)
            return carry

        lax.fori_loop(0, nblk // _ISLOT, round_body, 0)
        plsc.subcore_barrier()
        pltpu.sync_copy(acc_sh.at[pl.ds(s * _RPT, _RPT)],
                        out_hbm.at[c].at[pl.ds(s * _RPT, _RPT)])

    return agg


def _prep_edges(idx2, nblk, nblk_pad):
    total = _NW * nblk * _BLK
    pad = total - idx2.shape[1]
    src = jnp.concatenate([idx2[0], jnp.zeros((pad,), idx2.dtype)])
    dst = jnp.concatenate([idx2[1], jnp.full((pad,), _N, idx2.dtype)])
    src = src.reshape(_NW, nblk, _BLK).astype(jnp.int32)
    dst = dst.reshape(_NW, nblk, _BLK).astype(jnp.int32)
    bp = nblk_pad - nblk
    src = jnp.pad(src, ((0, 0), (0, bp), (0, 0)))
    dst = jnp.pad(dst, ((0, 0), (0, bp), (0, 0)), constant_values=_N)
    return src, dst


def _emb_body(x_ref, w_ref, b_ref, o_ref):
    o_ref[...] = (lax.dot_general(x_ref[...], w_ref[...], _MM,
                                  preferred_element_type=jnp.float32,
                                  precision=_PREC) + b_ref[...])


def _emb(x, w, b):
    return pl.pallas_call(
        _emb_body,
        grid=(_GRID,),
        in_specs=[
            pl.BlockSpec((_RB, _H), lambda i: (i, 0)),
            pl.BlockSpec((_H, _H), lambda i: (0, 0)),
            pl.BlockSpec((1, _H), lambda i: (0, 0)),
        ],
        out_specs=pl.BlockSpec((_RB, _H), lambda i: (i, 0)),
        out_shape=jax.ShapeDtypeStruct((_N, _H), jnp.float32),
    )(x, w, b.reshape(1, _H))


def _mlp_body(h_ref, a0_ref, a1_ref, w1_ref, b1_ref, w2_ref, b2_ref,
              u_ref, s1_ref, s2_ref):
    i = pl.program_id(0)
    t = h_ref[...] + a0_ref[0] + a1_ref[0]
    t = jnp.maximum(lax.dot_general(t, w1_ref[...], _MM,
                                    preferred_element_type=jnp.float32,
                                    precision=_PREC) + b1_ref[...], 0.0)
    u = (lax.dot_general(t, w2_ref[...], _MM,
                         preferred_element_type=jnp.float32,
                         precision=_PREC) + b2_ref[...])
    u_ref[...] = u
    ps1 = jnp.sum(u, axis=0, keepdims=True)
    ps2 = jnp.sum(u * u, axis=0, keepdims=True)

    @pl.when(i == 0)
    def _():
        s1_ref[...] = ps1
        s2_ref[...] = ps2

    @pl.when(i > 0)
    def _():
        s1_ref[...] += ps1
        s2_ref[...] += ps2


def _mlp(h, agg, w1, b1, w2, b2):
    return pl.pallas_call(
        _mlp_body,
        grid=(_GRID,),
        in_specs=[
            pl.BlockSpec((_RB, _H), lambda i: (i, 0)),
            pl.BlockSpec((1, _RB, _H), lambda i: (0, i, 0)),
            pl.BlockSpec((1, _RB, _H), lambda i: (1, i, 0)),
            pl.BlockSpec((_H, _H), lambda i: (0, 0)),
            pl.BlockSpec((1, _H), lambda i: (0, 0)),
            pl.BlockSpec((_H, _H), lambda i: (0, 0)),
            pl.BlockSpec((1, _H), lambda i: (0, 0)),
        ],
        out_specs=[
            pl.BlockSpec((_RB, _H), lambda i: (i, 0)),
            pl.BlockSpec((1, _H), lambda i: (0, 0)),
            pl.BlockSpec((1, _H), lambda i: (0, 0)),
        ],
        out_shape=[
            jax.ShapeDtypeStruct((_N, _H), jnp.float32),
            jax.ShapeDtypeStruct((1, _H), jnp.float32),
            jax.ShapeDtypeStruct((1, _H), jnp.float32),
        ],
    )(h, agg, agg, w1, b1.reshape(1, _H), w2, b2.reshape(1, _H))


def _bnres_body(u_ref, s1_ref, s2_ref, g_ref, b_ref, h_ref, batch_ref,
                hn_ref, pool_ref):
    i = pl.program_id(0)
    m = s1_ref[...] / _N
    v = s2_ref[...] / _N - m * m
    inv = lax.rsqrt(v + 1e-5)
    t = (u_ref[...] - m) * inv * g_ref[...] + b_ref[...]
    hn = jnp.maximum(t, 0.0) + h_ref[...]
    hn_ref[...] = hn
    onehot = (batch_ref[...] ==
              lax.broadcasted_iota(jnp.int32, (_RB, _G), 1)).astype(jnp.float32)
    pp = lax.dot_general(onehot, hn, (((0,), (0,)), ((), ())),
                         preferred_element_type=jnp.float32, precision=_PREC)

    @pl.when(i == 0)
    def _():
        pool_ref[...] = pp

    @pl.when(i > 0)
    def _():
        pool_ref[...] += pp


def _bnres(u, s1, s2, g, b, h, batch2):
    return pl.pallas_call(
        _bnres_body,
        grid=(_GRID,),
        in_specs=[
            pl.BlockSpec((_RB, _H), lambda i: (i, 0)),
            pl.BlockSpec((1, _H), lambda i: (0, 0)),
            pl.BlockSpec((1, _H), lambda i: (0, 0)),
            pl.BlockSpec((1, _H), lambda i: (0, 0)),
            pl.BlockSpec((1, _H), lambda i: (0, 0)),
            pl.BlockSpec((_RB, _H), lambda i: (i, 0)),
            pl.BlockSpec((_RB, 1), lambda i: (i, 0)),
        ],
        out_specs=[
            pl.BlockSpec((_RB, _H), lambda i: (i, 0)),
            pl.BlockSpec((_G, _H), lambda i: (0, 0)),
        ],
        out_shape=[
            jax.ShapeDtypeStruct((_N, _H), jnp.float32),
            jax.ShapeDtypeStruct((_G, _H), jnp.float32),
        ],
    )(u, s1, s2, g.reshape(1, _H), b.reshape(1, _H), h, batch2)


def _final_body(p_ref, w_ref, b_ref, o_ref):
    acc = jnp.zeros((_G, _OUT), jnp.float32)
    for i in range(_NL + 1):
        acc = acc + lax.dot_general(p_ref[i], w_ref[i], _MM,
                                    preferred_element_type=jnp.float32,
                                    precision=_PREC)
    o_ref[...] = acc + jnp.sum(b_ref[...], axis=0, keepdims=True)


def _final(pools, w, b):
    return pl.pallas_call(
        _final_body,
        out_shape=jax.ShapeDtypeStruct((_G, _OUT), jnp.float32),
    )(pools, w, b)


def kernel(x, edge_index, cycle_index, batch, params):
    p = params

    def _nblk(num_edges):
        per_tile = -(-num_edges // _NW)
        nb = -(-per_tile // _BLK)
        return -(-nb // _ISLOT) * _ISLOT

    nblk_e = _nblk(edge_index.shape[1])
    nblk_c = _nblk(cycle_index.shape[1])
    agg_e = _make_agg(nblk_e)
    agg_c = _make_agg(nblk_c)
    esrc, edst = _prep_edges(edge_index, nblk_e, nblk_e)
    csrc, cdst = _prep_edges(cycle_index, nblk_c, nblk_c)
    zeros = jnp.zeros((_NPAD, _H), jnp.float32)
    batch2 = batch.astype(jnp.int32).reshape(_N, 1)

    x0 = _emb(x, p["emb_w"], p["emb_b"])

    cagg = agg_c(x0, csrc, cdst, zeros)

    pools = []
    h = x0
    for i in range(_NL):
        eagg = agg_e(h, esrc, edst, zeros)
        u, s1, s2 = _mlp(h, eagg, p["conv_w1"][i], p["conv_b1"][i],
                         p["conv_w2"][i], p["conv_b2"][i])
        h, pool = _bnres(u, s1, s2, p["bn_g"][i], p["bn_b"][i], h, batch2)
        pools.append(pool)

    u, s1, s2 = _mlp(x0, cagg, p["conv2_w1"], p["conv2_b1"],
                     p["conv2_w2"], p["conv2_b2"])
    h4, pool4 = _bnres(u, s1, s2, p["bn2_g"], p["bn2_b"], x0, batch2)
    pools.append(pool4)

    return _final(jnp.stack(pools), p["lin_w"], p["lin_b"])


# speedup vs baseline: 4.2501x; 4.2501x over previous
import functools

import jax
import jax.numpy as jnp
from jax import lax
from jax.experimental import pallas as pl
from jax.experimental.pallas import tpu as pltpu
from jax.experimental.pallas import tpu_sc as plsc

_N = 10000
_H = 128
_G = 64
_OUT = 64
_NL = 3

_NC = 2
_NS = 16
_NW = _NC * _NS
_BLK = 128
_NPAD = _N + 112
_RPT = _NPAD // _NS

_RB = 1000
_GRID = _N // _RB

_MM = (((1,), (0,)), ((), ()))
_PREC = lax.Precision.HIGHEST


_NBUF = 2
_ISLOT = 2 * _NBUF
_NBLK_PAD = 80


def _make_agg(nblk):
    mesh = plsc.VectorSubcoreMesh(core_axis_name="c", subcore_axis_name="s",
                                  num_cores=_NC, num_subcores=_NS)

    @functools.partial(
        pl.kernel,
        mesh=mesh,
        out_type=jax.ShapeDtypeStruct((_NC, _NPAD, _H), jnp.float32),
        scratch_types=[
            pltpu.VMEM((_ISLOT, _BLK), jnp.int32),
            pltpu.VMEM((_ISLOT, _BLK), jnp.int32),
            pltpu.VMEM((_NBUF, _BLK, _H), jnp.float32),
            pltpu.VMEM_SHARED((_NPAD, _H), jnp.float32),
        ] + [pltpu.SemaphoreType.DMA] * (_NBUF + _ISLOT),
    )
    def agg(h_hbm, src_hbm, dst_hbm, zeros_hbm, out_hbm,
            src_r, dst_r, rows_v, acc_sh, *sems):
        gsems = sems[:_NBUF]
        isems = sems[_NBUF:]
        c = lax.axis_index("c")
        s = lax.axis_index("s")
        wid = c * _NS + s
        pltpu.sync_copy(zeros_hbm.at[pl.ds(s * _RPT, _RPT)],
                        acc_sh.at[pl.ds(s * _RPT, _RPT)])
        plsc.subcore_barrier()

        def fetch_idx(j, slot):
            pltpu.async_copy(src_hbm.at[wid].at[j], src_r.at[slot],
                             isems[slot])
            pltpu.async_copy(dst_hbm.at[wid].at[j], dst_r.at[slot],
                             isems[slot])

        def wait_idx(j, slot):
            pltpu.make_async_copy(src_hbm.at[wid].at[j], src_r.at[slot],
                                  isems[slot]).wait()
            pltpu.make_async_copy(dst_hbm.at[wid].at[j], dst_r.at[slot],
                                  isems[slot]).wait()

        def fetch_rows(slot, b):
            pltpu.async_copy(h_hbm.at[src_r.at[slot]], rows_v.at[b], gsems[b])

        for t in range(_ISLOT):
            fetch_idx(t, t)
        for b in range(_NBUF):
            wait_idx(b, b)
            fetch_rows(b, b)

        def round_body(r, carry):
            j0 = r * _ISLOT
            for t in range(_ISLOT):
                j = j0 + t
                slot = t
                b = t % _NBUF
                pltpu.make_async_copy(h_hbm.at[src_r.at[slot]], rows_v.at[b],
                                      gsems[b]).wait()
                pltpu.sync_copy(rows_v.at[b], acc_sh.at[dst_r.at[slot]],
                                add=True)

                @pl.when(j + _ISLOT < nblk)
                def _():
                    fetch_idx(j + _ISLOT, slot)

                jn = j + _NBUF
                sn = (t + _NBUF) % _ISLOT

                @pl.when(jn < nblk)
                def _():
                    wait_idx(jn, sn)
                    fetch_rows(sn, b)
            return carry

        lax.fori_loop(0, nblk // _ISLOT, round_body, 0)
        plsc.subcore_barrier()
        pltpu.sync_copy(acc_sh.at[pl.ds(s * _RPT, _RPT)],
                        out_hbm.at[c].at[pl.ds(s * _RPT, _RPT)])

    return agg


def _prep_edges(idx2, nblk, nblk_pad):
    total = _NW * nblk * _BLK
    pad = total - idx2.shape[1]
    pad_i = jnp.arange(pad, dtype=jnp.int32)
    src = jnp.concatenate([idx2[0].astype(jnp.int32), (pad_i * 97) % _N])
    dst = jnp.concatenate([idx2[1].astype(jnp.int32),
                           _N + (pad_i % (_NPAD - _N))])
    src = src.reshape(_NW, nblk, _BLK)
    dst = dst.reshape(_NW, nblk, _BLK)
    bp = nblk_pad - nblk
    if bp:
        blk_i = jnp.arange(_NW * bp * _BLK, dtype=jnp.int32)
        src = jnp.concatenate(
            [src, ((blk_i * 89) % _N).reshape(_NW, bp, _BLK)], axis=1)
        dst = jnp.concatenate(
            [dst, (_N + (blk_i % (_NPAD - _N))).reshape(_NW, bp, _BLK)],
            axis=1)
    return src, dst


def _emb_body(x_ref, w_ref, b_ref, o_ref):
    o_ref[...] = (lax.dot_general(x_ref[...], w_ref[...], _MM,
                                  preferred_element_type=jnp.float32,
                                  precision=_PREC) + b_ref[...])


def _emb(x, w, b):
    return pl.pallas_call(
        _emb_body,
        grid=(_GRID,),
        in_specs=[
            pl.BlockSpec((_RB, _H), lambda i: (i, 0)),
            pl.BlockSpec((_H, _H), lambda i: (0, 0)),
            pl.BlockSpec((1, _H), lambda i: (0, 0)),
        ],
        out_specs=pl.BlockSpec((_RB, _H), lambda i: (i, 0)),
        out_shape=jax.ShapeDtypeStruct((_N, _H), jnp.float32),
    )(x, w, b.reshape(1, _H))


def _mlp_body(h_ref, a0_ref, a1_ref, w1_ref, b1_ref, w2_ref, b2_ref,
              u_ref, s1_ref, s2_ref):
    i = pl.program_id(0)
    t = h_ref[...] + a0_ref[0] + a1_ref[0]
    t = jnp.maximum(lax.dot_general(t, w1_ref[...], _MM,
                                    preferred_element_type=jnp.float32,
                                    precision=_PREC) + b1_ref[...], 0.0)
    u = (lax.dot_general(t, w2_ref[...], _MM,
                         preferred_element_type=jnp.float32,
                         precision=_PREC) + b2_ref[...])
    u_ref[...] = u
    ps1 = jnp.sum(u, axis=0, keepdims=True)
    ps2 = jnp.sum(u * u, axis=0, keepdims=True)

    @pl.when(i == 0)
    def _():
        s1_ref[...] = ps1
        s2_ref[...] = ps2

    @pl.when(i > 0)
    def _():
        s1_ref[...] += ps1
        s2_ref[...] += ps2


def _mlp(h, agg, w1, b1, w2, b2):
    return pl.pallas_call(
        _mlp_body,
        grid=(_GRID,),
        in_specs=[
            pl.BlockSpec((_RB, _H), lambda i: (i, 0)),
            pl.BlockSpec((1, _RB, _H), lambda i: (0, i, 0)),
            pl.BlockSpec((1, _RB, _H), lambda i: (1, i, 0)),
            pl.BlockSpec((_H, _H), lambda i: (0, 0)),
            pl.BlockSpec((1, _H), lambda i: (0, 0)),
            pl.BlockSpec((_H, _H), lambda i: (0, 0)),
            pl.BlockSpec((1, _H), lambda i: (0, 0)),
        ],
        out_specs=[
            pl.BlockSpec((_RB, _H), lambda i: (i, 0)),
            pl.BlockSpec((1, _H), lambda i: (0, 0)),
            pl.BlockSpec((1, _H), lambda i: (0, 0)),
        ],
        out_shape=[
            jax.ShapeDtypeStruct((_N, _H), jnp.float32),
            jax.ShapeDtypeStruct((1, _H), jnp.float32),
            jax.ShapeDtypeStruct((1, _H), jnp.float32),
        ],
    )(h, agg, agg, w1, b1.reshape(1, _H), w2, b2.reshape(1, _H))


def _bnres_body(u_ref, s1_ref, s2_ref, g_ref, b_ref, h_ref, batch_ref,
                hn_ref, pool_ref):
    i = pl.program_id(0)
    m = s1_ref[...] / _N
    v = s2_ref[...] / _N - m * m
    inv = lax.rsqrt(v + 1e-5)
    t = (u_ref[...] - m) * inv * g_ref[...] + b_ref[...]
    hn = jnp.maximum(t, 0.0) + h_ref[...]
    hn_ref[...] = hn
    onehot = (batch_ref[...] ==
              lax.broadcasted_iota(jnp.int32, (_RB, _G), 1)).astype(jnp.float32)
    pp = lax.dot_general(onehot, hn, (((0,), (0,)), ((), ())),
                         preferred_element_type=jnp.float32, precision=_PREC)

    @pl.when(i == 0)
    def _():
        pool_ref[...] = pp

    @pl.when(i > 0)
    def _():
        pool_ref[...] += pp


def _bnres(u, s1, s2, g, b, h, batch2):
    return pl.pallas_call(
        _bnres_body,
        grid=(_GRID,),
        in_specs=[
            pl.BlockSpec((_RB, _H), lambda i: (i, 0)),
            pl.BlockSpec((1, _H), lambda i: (0, 0)),
            pl.BlockSpec((1, _H), lambda i: (0, 0)),
            pl.BlockSpec((1, _H), lambda i: (0, 0)),
            pl.BlockSpec((1, _H), lambda i: (0, 0)),
            pl.BlockSpec((_RB, _H), lambda i: (i, 0)),
            pl.BlockSpec((_RB, 1), lambda i: (i, 0)),
        ],
        out_specs=[
            pl.BlockSpec((_RB, _H), lambda i: (i, 0)),
            pl.BlockSpec((_G, _H), lambda i: (0, 0)),
        ],
        out_shape=[
            jax.ShapeDtypeStruct((_N, _H), jnp.float32),
            jax.ShapeDtypeStruct((_G, _H), jnp.float32),
        ],
    )(u, s1, s2, g.reshape(1, _H), b.reshape(1, _H), h, batch2)


def _final_body(p_ref, w_ref, b_ref, o_ref):
    acc = jnp.zeros((_G, _OUT), jnp.float32)
    for i in range(_NL + 1):
        acc = acc + lax.dot_general(p_ref[i], w_ref[i], _MM,
                                    preferred_element_type=jnp.float32,
                                    precision=_PREC)
    o_ref[...] = acc + jnp.sum(b_ref[...], axis=0, keepdims=True)


def _final(pools, w, b):
    return pl.pallas_call(
        _final_body,
        out_shape=jax.ShapeDtypeStruct((_G, _OUT), jnp.float32),
    )(pools, w, b)


def kernel(x, edge_index, cycle_index, batch, params):
    p = params

    def _nblk(num_edges):
        per_tile = -(-num_edges // _NW)
        nb = -(-per_tile // _BLK)
        return -(-nb // _ISLOT) * _ISLOT

    nblk_e = _nblk(edge_index.shape[1])
    nblk_c = _nblk(cycle_index.shape[1])
    agg_e = _make_agg(nblk_e)
    agg_c = _make_agg(nblk_c)
    esrc, edst = _prep_edges(edge_index, nblk_e, nblk_e)
    csrc, cdst = _prep_edges(cycle_index, nblk_c, nblk_c)
    zeros = jnp.zeros((_NPAD, _H), jnp.float32)
    batch2 = batch.astype(jnp.int32).reshape(_N, 1)

    x0 = _emb(x, p["emb_w"], p["emb_b"])

    cagg = agg_c(x0, csrc, cdst, zeros)

    pools = []
    h = x0
    for i in range(_NL):
        eagg = agg_e(h, esrc, edst, zeros)
        u, s1, s2 = _mlp(h, eagg, p["conv_w1"][i], p["conv_b1"][i],
                         p["conv_w2"][i], p["conv_b2"][i])
        h, pool = _bnres(u, s1, s2, p["bn_g"][i], p["bn_b"][i], h, batch2)
        pools.append(pool)

    u, s1, s2 = _mlp(x0, cagg, p["conv2_w1"], p["conv2_b1"],
                     p["conv2_w2"], p["conv2_b2"])
    h4, pool4 = _bnres(u, s1, s2, p["bn2_g"], p["bn2_b"], x0, batch2)
    pools.append(pool4)

    return _final(jnp.stack(pools), p["lin_w"], p["lin_b"])


# speedup vs baseline: 4.6981x; 1.1054x over previous
import functools

import jax
import jax.numpy as jnp
from jax import lax
from jax.experimental import pallas as pl
from jax.experimental.pallas import tpu as pltpu
from jax.experimental.pallas import tpu_sc as plsc

_N = 10000
_H = 128
_G = 64
_OUT = 64
_NL = 3

_NC = 2
_NS = 16
_NW = _NC * _NS
_BLK = 128
_NPAD = _N + 112
_RPT = _NPAD // _NS

_RB = 1000
_GRID = _N // _RB

_MM = (((1,), (0,)), ((), ()))
_PREC = None


_NBUF = 2
_ISLOT = 2 * _NBUF
_NBLK_PAD = 80


def _make_agg(nblk):
    mesh = plsc.VectorSubcoreMesh(core_axis_name="c", subcore_axis_name="s",
                                  num_cores=_NC, num_subcores=_NS)

    @functools.partial(
        pl.kernel,
        mesh=mesh,
        out_type=jax.ShapeDtypeStruct((_NC, _NPAD, _H), jnp.float32),
        scratch_types=[
            pltpu.VMEM((_ISLOT, _BLK), jnp.int32),
            pltpu.VMEM((_ISLOT, _BLK), jnp.int32),
            pltpu.VMEM((_NBUF, _BLK, _H), jnp.float32),
            pltpu.VMEM_SHARED((_NPAD, _H), jnp.float32),
        ] + [pltpu.SemaphoreType.DMA] * (_NBUF + _ISLOT),
    )
    def agg(h_hbm, src_hbm, dst_hbm, zeros_hbm, out_hbm,
            src_r, dst_r, rows_v, acc_sh, *sems):
        gsems = sems[:_NBUF]
        isems = sems[_NBUF:]
        c = lax.axis_index("c")
        s = lax.axis_index("s")
        wid = c * _NS + s
        pltpu.sync_copy(zeros_hbm.at[pl.ds(s * _RPT, _RPT)],
                        acc_sh.at[pl.ds(s * _RPT, _RPT)])
        plsc.subcore_barrier()

        def fetch_idx(j, slot):
            pltpu.async_copy(src_hbm.at[wid].at[j], src_r.at[slot],
                             isems[slot])
            pltpu.async_copy(dst_hbm.at[wid].at[j], dst_r.at[slot],
                             isems[slot])

        def wait_idx(j, slot):
            pltpu.make_async_copy(src_hbm.at[wid].at[j], src_r.at[slot],
                                  isems[slot]).wait()
            pltpu.make_async_copy(dst_hbm.at[wid].at[j], dst_r.at[slot],
                                  isems[slot]).wait()

        def fetch_rows(slot, b):
            pltpu.async_copy(h_hbm.at[src_r.at[slot]], rows_v.at[b], gsems[b])

        for t in range(_ISLOT):
            fetch_idx(t, t)
        for b in range(_NBUF):
            wait_idx(b, b)
            fetch_rows(b, b)

        def round_body(r, carry):
            j0 = r * _ISLOT
            for t in range(_ISLOT):
                j = j0 + t
                slot = t
                b = t % _NBUF
                pltpu.make_async_copy(h_hbm.at[src_r.at[slot]], rows_v.at[b],
                                      gsems[b]).wait()
                pltpu.sync_copy(rows_v.at[b], acc_sh.at[dst_r.at[slot]],
                                add=True)

                @pl.when(j + _ISLOT < nblk)
                def _():
                    fetch_idx(j + _ISLOT, slot)

                jn = j + _NBUF
                sn = (t + _NBUF) % _ISLOT

                @pl.when(jn < nblk)
                def _():
                    wait_idx(jn, sn)
                    fetch_rows(sn, b)
            return carry

        lax.fori_loop(0, nblk // _ISLOT, round_body, 0)
        plsc.subcore_barrier()
        pltpu.sync_copy(acc_sh.at[pl.ds(s * _RPT, _RPT)],
                        out_hbm.at[c].at[pl.ds(s * _RPT, _RPT)])

    return agg


def _prep_edges(idx2, nblk, nblk_pad):
    total = _NW * nblk * _BLK
    pad = total - idx2.shape[1]
    pad_i = jnp.arange(pad, dtype=jnp.int32)
    src = jnp.concatenate([idx2[0].astype(jnp.int32), (pad_i * 97) % _N])
    dst = jnp.concatenate([idx2[1].astype(jnp.int32),
                           _N + (pad_i % (_NPAD - _N))])
    src = src.reshape(_NW, nblk, _BLK)
    dst = dst.reshape(_NW, nblk, _BLK)
    bp = nblk_pad - nblk
    if bp:
        blk_i = jnp.arange(_NW * bp * _BLK, dtype=jnp.int32)
        src = jnp.concatenate(
            [src, ((blk_i * 89) % _N).reshape(_NW, bp, _BLK)], axis=1)
        dst = jnp.concatenate(
            [dst, (_N + (blk_i % (_NPAD - _N))).reshape(_NW, bp, _BLK)],
            axis=1)
    return src, dst


def _emb_body(x_ref, w_ref, b_ref, o_ref):
    o_ref[...] = (lax.dot_general(x_ref[...], w_ref[...], _MM,
                                  preferred_element_type=jnp.float32,
                                  precision=_PREC) + b_ref[...])


def _emb(x, w, b):
    return pl.pallas_call(
        _emb_body,
        grid=(_GRID,),
        in_specs=[
            pl.BlockSpec((_RB, _H), lambda i: (i, 0)),
            pl.BlockSpec((_H, _H), lambda i: (0, 0)),
            pl.BlockSpec((1, _H), lambda i: (0, 0)),
        ],
        out_specs=pl.BlockSpec((_RB, _H), lambda i: (i, 0)),
        out_shape=jax.ShapeDtypeStruct((_N, _H), jnp.float32),
    )(x, w, b.reshape(1, _H))


def _mlp_body(h_ref, a0_ref, a1_ref, w1_ref, b1_ref, w2_ref, b2_ref,
              u_ref, s1_ref, s2_ref):
    i = pl.program_id(0)
    t = h_ref[...] + a0_ref[0] + a1_ref[0]
    t = jnp.maximum(lax.dot_general(t, w1_ref[...], _MM,
                                    preferred_element_type=jnp.float32,
                                    precision=_PREC) + b1_ref[...], 0.0)
    u = (lax.dot_general(t, w2_ref[...], _MM,
                         preferred_element_type=jnp.float32,
                         precision=_PREC) + b2_ref[...])
    u_ref[...] = u
    ps1 = jnp.sum(u, axis=0, keepdims=True)
    ps2 = jnp.sum(u * u, axis=0, keepdims=True)

    @pl.when(i == 0)
    def _():
        s1_ref[...] = ps1
        s2_ref[...] = ps2

    @pl.when(i > 0)
    def _():
        s1_ref[...] += ps1
        s2_ref[...] += ps2


def _mlp(h, agg, w1, b1, w2, b2):
    return pl.pallas_call(
        _mlp_body,
        grid=(_GRID,),
        in_specs=[
            pl.BlockSpec((_RB, _H), lambda i: (i, 0)),
            pl.BlockSpec((1, _RB, _H), lambda i: (0, i, 0)),
            pl.BlockSpec((1, _RB, _H), lambda i: (1, i, 0)),
            pl.BlockSpec((_H, _H), lambda i: (0, 0)),
            pl.BlockSpec((1, _H), lambda i: (0, 0)),
            pl.BlockSpec((_H, _H), lambda i: (0, 0)),
            pl.BlockSpec((1, _H), lambda i: (0, 0)),
        ],
        out_specs=[
            pl.BlockSpec((_RB, _H), lambda i: (i, 0)),
            pl.BlockSpec((1, _H), lambda i: (0, 0)),
            pl.BlockSpec((1, _H), lambda i: (0, 0)),
        ],
        out_shape=[
            jax.ShapeDtypeStruct((_N, _H), jnp.float32),
            jax.ShapeDtypeStruct((1, _H), jnp.float32),
            jax.ShapeDtypeStruct((1, _H), jnp.float32),
        ],
    )(h, agg, agg, w1, b1.reshape(1, _H), w2, b2.reshape(1, _H))


def _bnres_body(u_ref, s1_ref, s2_ref, g_ref, b_ref, h_ref, batch_ref,
                hn_ref, pool_ref):
    i = pl.program_id(0)
    m = s1_ref[...] / _N
    v = s2_ref[...] / _N - m * m
    inv = lax.rsqrt(v + 1e-5)
    t = (u_ref[...] - m) * inv * g_ref[...] + b_ref[...]
    hn = jnp.maximum(t, 0.0) + h_ref[...]
    hn_ref[...] = hn
    onehot = (batch_ref[...] ==
              lax.broadcasted_iota(jnp.int32, (_RB, _G), 1)).astype(jnp.float32)
    pp = lax.dot_general(onehot, hn, (((0,), (0,)), ((), ())),
                         preferred_element_type=jnp.float32, precision=_PREC)

    @pl.when(i == 0)
    def _():
        pool_ref[...] = pp

    @pl.when(i > 0)
    def _():
        pool_ref[...] += pp


def _bnres(u, s1, s2, g, b, h, batch2):
    return pl.pallas_call(
        _bnres_body,
        grid=(_GRID,),
        in_specs=[
            pl.BlockSpec((_RB, _H), lambda i: (i, 0)),
            pl.BlockSpec((1, _H), lambda i: (0, 0)),
            pl.BlockSpec((1, _H), lambda i: (0, 0)),
            pl.BlockSpec((1, _H), lambda i: (0, 0)),
            pl.BlockSpec((1, _H), lambda i: (0, 0)),
            pl.BlockSpec((_RB, _H), lambda i: (i, 0)),
            pl.BlockSpec((_RB, 1), lambda i: (i, 0)),
        ],
        out_specs=[
            pl.BlockSpec((_RB, _H), lambda i: (i, 0)),
            pl.BlockSpec((_G, _H), lambda i: (0, 0)),
        ],
        out_shape=[
            jax.ShapeDtypeStruct((_N, _H), jnp.float32),
            jax.ShapeDtypeStruct((_G, _H), jnp.float32),
        ],
    )(u, s1, s2, g.reshape(1, _H), b.reshape(1, _H), h, batch2)


def _final_body(p_ref, w_ref, b_ref, o_ref):
    acc = jnp.zeros((_G, _OUT), jnp.float32)
    for i in range(_NL + 1):
        acc = acc + lax.dot_general(p_ref[i], w_ref[i], _MM,
                                    preferred_element_type=jnp.float32,
                                    precision=_PREC)
    o_ref[...] = acc + jnp.sum(b_ref[...], axis=0, keepdims=True)


def _final(pools, w, b):
    return pl.pallas_call(
        _final_body,
        out_shape=jax.ShapeDtypeStruct((_G, _OUT), jnp.float32),
    )(pools, w, b)


def kernel(x, edge_index, cycle_index, batch, params):
    p = params

    def _nblk(num_edges):
        per_tile = -(-num_edges // _NW)
        nb = -(-per_tile // _BLK)
        return -(-nb // _ISLOT) * _ISLOT

    nblk_e = _nblk(edge_index.shape[1])
    nblk_c = _nblk(cycle_index.shape[1])
    agg_e = _make_agg(nblk_e)
    agg_c = _make_agg(nblk_c)
    esrc, edst = _prep_edges(edge_index, nblk_e, nblk_e)
    csrc, cdst = _prep_edges(cycle_index, nblk_c, nblk_c)
    zeros = jnp.zeros((_NPAD, _H), jnp.float32)
    batch2 = batch.astype(jnp.int32).reshape(_N, 1)

    x0 = _emb(x, p["emb_w"], p["emb_b"])

    cagg = agg_c(x0, csrc, cdst, zeros)

    pools = []
    h = x0
    for i in range(_NL):
        eagg = agg_e(h, esrc, edst, zeros)
        u, s1, s2 = _mlp(h, eagg, p["conv_w1"][i], p["conv_b1"][i],
                         p["conv_w2"][i], p["conv_b2"][i])
        h, pool = _bnres(u, s1, s2, p["bn_g"][i], p["bn_b"][i], h, batch2)
        pools.append(pool)

    u, s1, s2 = _mlp(x0, cagg, p["conv2_w1"], p["conv2_b1"],
                     p["conv2_w2"], p["conv2_b2"])
    h4, pool4 = _bnres(u, s1, s2, p["bn2_g"], p["bn2_b"], x0, batch2)
    pools.append(pool4)

    return _final(jnp.stack(pools), p["lin_w"], p["lin_b"])


# speedup vs baseline: 4.8146x; 1.0248x over previous
import functools

import jax
import jax.numpy as jnp
from jax import lax
from jax.experimental import pallas as pl
from jax.experimental.pallas import tpu as pltpu
from jax.experimental.pallas import tpu_sc as plsc

_N = 10000
_H = 128
_G = 64
_OUT = 64
_NL = 3

_NC = 2
_NS = 16
_NW = _NC * _NS
_BLK = 128
_NPAD = _N + 112
_RPT = _NPAD // _NS

_RB = 1000
_GRID = _N // _RB

_MM = (((1,), (0,)), ((), ()))
_PREC = None


_NBUF = 2
_ISLOT = 2 * _NBUF
_NBLK_PAD = 80


def _make_agg(nblk):
    mesh = plsc.VectorSubcoreMesh(core_axis_name="c", subcore_axis_name="s",
                                  num_cores=_NC, num_subcores=_NS)

    @functools.partial(
        pl.kernel,
        mesh=mesh,
        out_type=jax.ShapeDtypeStruct((_NC, _NPAD, _H), jnp.float32),
        scratch_types=[
            pltpu.VMEM((_ISLOT, _BLK), jnp.int32),
            pltpu.VMEM((_ISLOT, _BLK), jnp.int32),
            pltpu.VMEM((_NBUF, _BLK, _H), jnp.float32),
            pltpu.VMEM_SHARED((_NPAD, _H), jnp.float32),
        ] + [pltpu.SemaphoreType.DMA] * (_NBUF + _ISLOT),
    )
    def agg(h_hbm, src_hbm, dst_hbm, zeros_hbm, out_hbm,
            src_r, dst_r, rows_v, acc_sh, *sems):
        gsems = sems[:_NBUF]
        isems = sems[_NBUF:]
        c = lax.axis_index("c")
        s = lax.axis_index("s")
        wid = c * _NS + s
        pltpu.sync_copy(zeros_hbm.at[pl.ds(s * _RPT, _RPT)],
                        acc_sh.at[pl.ds(s * _RPT, _RPT)])
        plsc.subcore_barrier()

        def fetch_idx(j, slot):
            pltpu.async_copy(src_hbm.at[wid].at[j], src_r.at[slot],
                             isems[slot])
            pltpu.async_copy(dst_hbm.at[wid].at[j], dst_r.at[slot],
                             isems[slot])

        def wait_idx(j, slot):
            pltpu.make_async_copy(src_hbm.at[wid].at[j], src_r.at[slot],
                                  isems[slot]).wait()
            pltpu.make_async_copy(dst_hbm.at[wid].at[j], dst_r.at[slot],
                                  isems[slot]).wait()

        def fetch_rows(slot, b):
            pltpu.async_copy(h_hbm.at[src_r.at[slot]], rows_v.at[b], gsems[b])

        for t in range(_ISLOT):
            fetch_idx(t, t)
        for b in range(_NBUF):
            wait_idx(b, b)
            fetch_rows(b, b)

        def round_body(r, carry):
            j0 = r * _ISLOT
            for t in range(_ISLOT):
                j = j0 + t
                slot = t
                b = t % _NBUF
                pltpu.make_async_copy(h_hbm.at[src_r.at[slot]], rows_v.at[b],
                                      gsems[b]).wait()
                pltpu.sync_copy(rows_v.at[b], acc_sh.at[dst_r.at[slot]],
                                add=True)

                @pl.when(j + _ISLOT < nblk)
                def _():
                    fetch_idx(j + _ISLOT, slot)

                jn = j + _NBUF
                sn = (t + _NBUF) % _ISLOT

                @pl.when(jn < nblk)
                def _():
                    wait_idx(jn, sn)
                    fetch_rows(sn, b)
            return carry

        lax.fori_loop(0, nblk // _ISLOT, round_body, 0)
        plsc.subcore_barrier()
        pltpu.sync_copy(acc_sh.at[pl.ds(s * _RPT, _RPT)],
                        out_hbm.at[c].at[pl.ds(s * _RPT, _RPT)])

    return agg


def _prep_edges(idx2, nblk, nblk_pad):
    total = _NW * nblk * _BLK
    pad = total - idx2.shape[1]
    pad_i = jnp.arange(pad, dtype=jnp.int32)
    src = jnp.concatenate([idx2[0].astype(jnp.int32), (pad_i * 97) % _N])
    dst = jnp.concatenate([idx2[1].astype(jnp.int32),
                           _N + (pad_i % (_NPAD - _N))])
    src = src.reshape(_NW, nblk, _BLK)
    dst = dst.reshape(_NW, nblk, _BLK)
    bp = nblk_pad - nblk
    if bp:
        blk_i = jnp.arange(_NW * bp * _BLK, dtype=jnp.int32)
        src = jnp.concatenate(
            [src, ((blk_i * 89) % _N).reshape(_NW, bp, _BLK)], axis=1)
        dst = jnp.concatenate(
            [dst, (_N + (blk_i % (_NPAD - _N))).reshape(_NW, bp, _BLK)],
            axis=1)
    return src, dst


def _emb_body(x_ref, w_ref, b_ref, o_ref):
    o_ref[...] = (lax.dot_general(x_ref[...], w_ref[...], _MM,
                                  preferred_element_type=jnp.float32,
                                  precision=_PREC) + b_ref[...])


def _emb(x, w, b):
    return pl.pallas_call(
        _emb_body,
        grid=(_GRID,),
        in_specs=[
            pl.BlockSpec((_RB, _H), lambda i: (i, 0)),
            pl.BlockSpec((_H, _H), lambda i: (0, 0)),
            pl.BlockSpec((1, _H), lambda i: (0, 0)),
        ],
        out_specs=pl.BlockSpec((_RB, _H), lambda i: (i, 0)),
        out_shape=jax.ShapeDtypeStruct((_N, _H), jnp.float32),
    )(x, w, b.reshape(1, _H))


def _layer_body(h_ref, a0_ref, a1_ref, w1_ref, b1_ref, w2_ref, b2_ref,
                g_ref, bb_ref, batch_ref, hn_ref, pool_ref,
                u_s, h_s, s1_s, s2_s):
    p = pl.program_id(0)
    i = pl.program_id(1)

    @pl.when(p == 0)
    def _():
        hb = h_ref[...]
        t = hb + a0_ref[0] + a1_ref[0]
        t = jnp.maximum(lax.dot_general(t, w1_ref[...], _MM,
                                        preferred_element_type=jnp.float32,
                                        precision=_PREC) + b1_ref[...], 0.0)
        u = (lax.dot_general(t, w2_ref[...], _MM,
                             preferred_element_type=jnp.float32,
                             precision=_PREC) + b2_ref[...])
        u_s[i] = u
        h_s[i] = hb
        ps1 = jnp.sum(u, axis=0, keepdims=True)
        ps2 = jnp.sum(u * u, axis=0, keepdims=True)

        @pl.when(i == 0)
        def _():
            s1_s[...] = ps1
            s2_s[...] = ps2

        @pl.when(i > 0)
        def _():
            s1_s[...] += ps1
            s2_s[...] += ps2

    @pl.when(p == 1)
    def _():
        m = s1_s[...] / _N
        v = s2_s[...] / _N - m * m
        inv = lax.rsqrt(v + 1e-5)
        t = (u_s[i] - m) * inv * g_ref[...] + bb_ref[...]
        hn = jnp.maximum(t, 0.0) + h_s[i]
        hn_ref[...] = hn
        onehot = (batch_ref[...] ==
                  lax.broadcasted_iota(jnp.int32, (_RB, _G), 1)
                  ).astype(jnp.float32)
        pp = lax.dot_general(onehot, hn, (((0,), (0,)), ((), ())),
                             preferred_element_type=jnp.float32,
                             precision=_PREC)

        @pl.when(i == 0)
        def _():
            pool_ref[...] = pp

        @pl.when(i > 0)
        def _():
            pool_ref[...] += pp


def _layer(h, agg, w1, b1, w2, b2, g, bb, batch2):
    return pl.pallas_call(
        _layer_body,
        grid=(2, _GRID),
        in_specs=[
            pl.BlockSpec((_RB, _H), lambda p, i: ((1 - p) * i, 0)),
            pl.BlockSpec((1, _RB, _H), lambda p, i: (0, (1 - p) * i, 0)),
            pl.BlockSpec((1, _RB, _H), lambda p, i: (1, (1 - p) * i, 0)),
            pl.BlockSpec((_H, _H), lambda p, i: (0, 0)),
            pl.BlockSpec((1, _H), lambda p, i: (0, 0)),
            pl.BlockSpec((_H, _H), lambda p, i: (0, 0)),
            pl.BlockSpec((1, _H), lambda p, i: (0, 0)),
            pl.BlockSpec((1, _H), lambda p, i: (0, 0)),
            pl.BlockSpec((1, _H), lambda p, i: (0, 0)),
            pl.BlockSpec((_RB, 1), lambda p, i: (p * i, 0)),
        ],
        out_specs=[
            pl.BlockSpec((_RB, _H), lambda p, i: (p * i, 0)),
            pl.BlockSpec((_G, _H), lambda p, i: (0, 0)),
        ],
        out_shape=[
            jax.ShapeDtypeStruct((_N, _H), jnp.float32),
            jax.ShapeDtypeStruct((_G, _H), jnp.float32),
        ],
        scratch_shapes=[
            pltpu.VMEM((_GRID, _RB, _H), jnp.float32),
            pltpu.VMEM((_GRID, _RB, _H), jnp.float32),
            pltpu.VMEM((1, _H), jnp.float32),
            pltpu.VMEM((1, _H), jnp.float32),
        ],
    )(h, agg, agg, w1, b1.reshape(1, _H), w2, b2.reshape(1, _H),
      g.reshape(1, _H), bb.reshape(1, _H), batch2)


def _mlp_body(h_ref, a0_ref, a1_ref, w1_ref, b1_ref, w2_ref, b2_ref,
              u_ref, s1_ref, s2_ref):
    i = pl.program_id(0)
    t = h_ref[...] + a0_ref[0] + a1_ref[0]
    t = jnp.maximum(lax.dot_general(t, w1_ref[...], _MM,
                                    preferred_element_type=jnp.float32,
                                    precision=_PREC) + b1_ref[...], 0.0)
    u = (lax.dot_general(t, w2_ref[...], _MM,
                         preferred_element_type=jnp.float32,
                         precision=_PREC) + b2_ref[...])
    u_ref[...] = u
    ps1 = jnp.sum(u, axis=0, keepdims=True)
    ps2 = jnp.sum(u * u, axis=0, keepdims=True)

    @pl.when(i == 0)
    def _():
        s1_ref[...] = ps1
        s2_ref[...] = ps2

    @pl.when(i > 0)
    def _():
        s1_ref[...] += ps1
        s2_ref[...] += ps2


def _mlp(h, agg, w1, b1, w2, b2):
    return pl.pallas_call(
        _mlp_body,
        grid=(_GRID,),
        in_specs=[
            pl.BlockSpec((_RB, _H), lambda i: (i, 0)),
            pl.BlockSpec((1, _RB, _H), lambda i: (0, i, 0)),
            pl.BlockSpec((1, _RB, _H), lambda i: (1, i, 0)),
            pl.BlockSpec((_H, _H), lambda i: (0, 0)),
            pl.BlockSpec((1, _H), lambda i: (0, 0)),
            pl.BlockSpec((_H, _H), lambda i: (0, 0)),
            pl.BlockSpec((1, _H), lambda i: (0, 0)),
        ],
        out_specs=[
            pl.BlockSpec((_RB, _H), lambda i: (i, 0)),
            pl.BlockSpec((1, _H), lambda i: (0, 0)),
            pl.BlockSpec((1, _H), lambda i: (0, 0)),
        ],
        out_shape=[
            jax.ShapeDtypeStruct((_N, _H), jnp.float32),
            jax.ShapeDtypeStruct((1, _H), jnp.float32),
            jax.ShapeDtypeStruct((1, _H), jnp.float32),
        ],
    )(h, agg, agg, w1, b1.reshape(1, _H), w2, b2.reshape(1, _H))


def _bnres_body(u_ref, s1_ref, s2_ref, g_ref, b_ref, h_ref, batch_ref,
                hn_ref, pool_ref):
    i = pl.program_id(0)
    m = s1_ref[...] / _N
    v = s2_ref[...] / _N - m * m
    inv = lax.rsqrt(v + 1e-5)
    t = (u_ref[...] - m) * inv * g_ref[...] + b_ref[...]
    hn = jnp.maximum(t, 0.0) + h_ref[...]
    hn_ref[...] = hn
    onehot = (batch_ref[...] ==
              lax.broadcasted_iota(jnp.int32, (_RB, _G), 1)).astype(jnp.float32)
    pp = lax.dot_general(onehot, hn, (((0,), (0,)), ((), ())),
                         preferred_element_type=jnp.float32, precision=_PREC)

    @pl.when(i == 0)
    def _():
        pool_ref[...] = pp

    @pl.when(i > 0)
    def _():
        pool_ref[...] += pp


def _bnres(u, s1, s2, g, b, h, batch2):
    return pl.pallas_call(
        _bnres_body,
        grid=(_GRID,),
        in_specs=[
            pl.BlockSpec((_RB, _H), lambda i: (i, 0)),
            pl.BlockSpec((1, _H), lambda i: (0, 0)),
            pl.BlockSpec((1, _H), lambda i: (0, 0)),
            pl.BlockSpec((1, _H), lambda i: (0, 0)),
            pl.BlockSpec((1, _H), lambda i: (0, 0)),
            pl.BlockSpec((_RB, _H), lambda i: (i, 0)),
            pl.BlockSpec((_RB, 1), lambda i: (i, 0)),
        ],
        out_specs=[
            pl.BlockSpec((_RB, _H), lambda i: (i, 0)),
            pl.BlockSpec((_G, _H), lambda i: (0, 0)),
        ],
        out_shape=[
            jax.ShapeDtypeStruct((_N, _H), jnp.float32),
            jax.ShapeDtypeStruct((_G, _H), jnp.float32),
        ],
    )(u, s1, s2, g.reshape(1, _H), b.reshape(1, _H), h, batch2)


def _final_body(p_ref, w_ref, b_ref, o_ref):
    acc = jnp.zeros((_G, _OUT), jnp.float32)
    for i in range(_NL + 1):
        acc = acc + lax.dot_general(p_ref[i], w_ref[i], _MM,
                                    preferred_element_type=jnp.float32,
                                    precision=_PREC)
    o_ref[...] = acc + jnp.sum(b_ref[...], axis=0, keepdims=True)


def _final(pools, w, b):
    return pl.pallas_call(
        _final_body,
        out_shape=jax.ShapeDtypeStruct((_G, _OUT), jnp.float32),
    )(pools, w, b)


def kernel(x, edge_index, cycle_index, batch, params):
    p = params

    def _nblk(num_edges):
        per_tile = -(-num_edges // _NW)
        nb = -(-per_tile // _BLK)
        return -(-nb // _ISLOT) * _ISLOT

    nblk_e = _nblk(edge_index.shape[1])
    nblk_c = _nblk(cycle_index.shape[1])
    agg_e = _make_agg(nblk_e)
    agg_c = _make_agg(nblk_c)
    esrc, edst = _prep_edges(edge_index, nblk_e, nblk_e)
    csrc, cdst = _prep_edges(cycle_index, nblk_c, nblk_c)
    zeros = jnp.zeros((_NPAD, _H), jnp.float32)
    batch2 = batch.astype(jnp.int32).reshape(_N, 1)

    x0 = _emb(x, p["emb_w"], p["emb_b"])

    cagg = agg_c(x0, csrc, cdst, zeros)

    pools = []
    h = x0
    for i in range(_NL):
        eagg = agg_e(h, esrc, edst, zeros)
        h, pool = _layer(h, eagg, p["conv_w1"][i], p["conv_b1"][i],
                         p["conv_w2"][i], p["conv_b2"][i],
                         p["bn_g"][i], p["bn_b"][i], batch2)
        pools.append(pool)

    h4, pool4 = _layer(x0, cagg, p["conv2_w1"], p["conv2_b1"],
                       p["conv2_w2"], p["conv2_b2"],
                       p["bn2_g"], p["bn2_b"], batch2)
    pools.append(pool4)

    return _final(jnp.stack(pools), p["lin_w"], p["lin_b"])


# speedup vs baseline: 4.9098x; 1.0198x over previous
import functools

import jax
import jax.numpy as jnp
from jax import lax
from jax.experimental import pallas as pl
from jax.experimental.pallas import tpu as pltpu
from jax.experimental.pallas import tpu_sc as plsc

_N = 10000
_H = 128
_G = 64
_OUT = 64
_NL = 3

_NC = 2
_NS = 16
_NW = _NC * _NS
_BLK = 128
_NPAD = _N + 112
_RPT = _NPAD // _NS

_RB = 1000
_GRID = _N // _RB

_MM = (((1,), (0,)), ((), ()))
_PREC = None


_NBUF = 2
_ISLOT = 2 * _NBUF
_NBLK_PAD = 80


def _make_agg(nblk):
    mesh = plsc.VectorSubcoreMesh(core_axis_name="c", subcore_axis_name="s",
                                  num_cores=_NC, num_subcores=_NS)

    @functools.partial(
        pl.kernel,
        mesh=mesh,
        out_type=jax.ShapeDtypeStruct((_NC, _NPAD, _H), jnp.float32),
        scratch_types=[
            pltpu.VMEM((_ISLOT, _BLK), jnp.int32),
            pltpu.VMEM((_ISLOT, _BLK), jnp.int32),
            pltpu.VMEM((_NBUF, _BLK, _H), jnp.float32),
            pltpu.VMEM_SHARED((_NPAD, _H), jnp.float32),
        ] + [pltpu.SemaphoreType.DMA] * (2 * _NBUF + _ISLOT + 1),
    )
    def agg(h_hbm, src_hbm, dst_hbm, zeros_hbm, out_hbm,
            src_r, dst_r, rows_v, acc_sh, *sems):
        gsems = sems[:_NBUF]
        ssems = sems[_NBUF:2 * _NBUF]
        isems = sems[2 * _NBUF:2 * _NBUF + _ISLOT]
        zsem = sems[-1]
        c = lax.axis_index("c")
        s = lax.axis_index("s")
        wid = c * _NS + s
        zdesc = pltpu.async_copy(zeros_hbm.at[pl.ds(s * _RPT, _RPT)],
                                 acc_sh.at[pl.ds(s * _RPT, _RPT)], zsem)

        def fetch_idx(j, slot):
            pltpu.async_copy(src_hbm.at[wid].at[j], src_r.at[slot],
                             isems[slot])
            pltpu.async_copy(dst_hbm.at[wid].at[j], dst_r.at[slot],
                             isems[slot])

        def wait_idx(j, slot):
            pltpu.make_async_copy(src_hbm.at[wid].at[j], src_r.at[slot],
                                  isems[slot]).wait()
            pltpu.make_async_copy(dst_hbm.at[wid].at[j], dst_r.at[slot],
                                  isems[slot]).wait()

        def fetch_rows(slot, b):
            pltpu.async_copy(h_hbm.at[src_r.at[slot]], rows_v.at[b], gsems[b])

        def wait_rows(slot, b):
            pltpu.make_async_copy(h_hbm.at[src_r.at[slot]], rows_v.at[b],
                                  gsems[b]).wait()

        def scatter_rows(slot, b):
            pltpu.async_copy(rows_v.at[b], acc_sh.at[dst_r.at[slot]],
                             ssems[b], add=True)

        def wait_scatter(slot, b):
            pltpu.make_async_copy(rows_v.at[b], acc_sh.at[dst_r.at[slot]],
                                  ssems[b]).wait()

        for t in range(_ISLOT):
            fetch_idx(t, t)
        for b in range(_NBUF):
            wait_idx(b, b)
            fetch_rows(b, b)
        zdesc.wait()
        plsc.subcore_barrier()

        def round_body(r, carry):
            j0 = r * _ISLOT
            for t in range(_ISLOT):
                j = j0 + t
                slot = t
                b = t % _NBUF
                jn = j + _NBUF
                sn = (t + _NBUF) % _ISLOT
                wait_rows(slot, b)
                scatter_rows(slot, b)

                @pl.when(jn < nblk)
                def _():
                    wait_idx(jn, sn)

                wait_scatter(slot, b)

                @pl.when(j + _ISLOT < nblk)
                def _():
                    fetch_idx(j + _ISLOT, slot)

                @pl.when(jn < nblk)
                def _():
                    fetch_rows(sn, b)
            return carry

        lax.fori_loop(0, nblk // _ISLOT, round_body, 0)
        plsc.subcore_barrier()
        pltpu.sync_copy(acc_sh.at[pl.ds(s * _RPT, _RPT)],
                        out_hbm.at[c].at[pl.ds(s * _RPT, _RPT)])

    return agg


def _prep_edges(idx2, nblk, nblk_pad):
    total = _NW * nblk * _BLK
    pad = total - idx2.shape[1]
    pad_i = jnp.arange(pad, dtype=jnp.int32)
    src = jnp.concatenate([idx2[0].astype(jnp.int32), (pad_i * 97) % _N])
    dst = jnp.concatenate([idx2[1].astype(jnp.int32),
                           _N + (pad_i % (_NPAD - _N))])
    src = src.reshape(_NW, nblk, _BLK)
    dst = dst.reshape(_NW, nblk, _BLK)
    bp = nblk_pad - nblk
    if bp:
        blk_i = jnp.arange(_NW * bp * _BLK, dtype=jnp.int32)
        src = jnp.concatenate(
            [src, ((blk_i * 89) % _N).reshape(_NW, bp, _BLK)], axis=1)
        dst = jnp.concatenate(
            [dst, (_N + (blk_i % (_NPAD - _N))).reshape(_NW, bp, _BLK)],
            axis=1)
    return src, dst


def _emb_body(x_ref, w_ref, b_ref, o_ref):
    o_ref[...] = (lax.dot_general(x_ref[...], w_ref[...], _MM,
                                  preferred_element_type=jnp.float32,
                                  precision=_PREC) + b_ref[...])


def _emb(x, w, b):
    return pl.pallas_call(
        _emb_body,
        grid=(_GRID,),
        in_specs=[
            pl.BlockSpec((_RB, _H), lambda i: (i, 0)),
            pl.BlockSpec((_H, _H), lambda i: (0, 0)),
            pl.BlockSpec((1, _H), lambda i: (0, 0)),
        ],
        out_specs=pl.BlockSpec((_RB, _H), lambda i: (i, 0)),
        out_shape=jax.ShapeDtypeStruct((_N, _H), jnp.float32),
    )(x, w, b.reshape(1, _H))


def _layer_body(h_ref, a0_ref, a1_ref, w1_ref, b1_ref, w2_ref, b2_ref,
                g_ref, bb_ref, batch_ref, hn_ref, pool_ref,
                u_s, h_s, s1_s, s2_s):
    p = pl.program_id(0)
    i = pl.program_id(1)

    @pl.when(p == 0)
    def _():
        hb = h_ref[...]
        t = hb + a0_ref[0] + a1_ref[0]
        t = jnp.maximum(lax.dot_general(t, w1_ref[...], _MM,
                                        preferred_element_type=jnp.float32,
                                        precision=_PREC) + b1_ref[...], 0.0)
        u = (lax.dot_general(t, w2_ref[...], _MM,
                             preferred_element_type=jnp.float32,
                             precision=_PREC) + b2_ref[...])
        u_s[i] = u
        h_s[i] = hb
        ps1 = jnp.sum(u, axis=0, keepdims=True)
        ps2 = jnp.sum(u * u, axis=0, keepdims=True)

        @pl.when(i == 0)
        def _():
            s1_s[...] = ps1
            s2_s[...] = ps2

        @pl.when(i > 0)
        def _():
            s1_s[...] += ps1
            s2_s[...] += ps2

    @pl.when(p == 1)
    def _():
        m = s1_s[...] / _N
        v = s2_s[...] / _N - m * m
        inv = lax.rsqrt(v + 1e-5)
        t = (u_s[i] - m) * inv * g_ref[...] + bb_ref[...]
        hn = jnp.maximum(t, 0.0) + h_s[i]
        hn_ref[...] = hn
        onehot = (batch_ref[...] ==
                  lax.broadcasted_iota(jnp.int32, (_RB, _G), 1)
                  ).astype(jnp.float32)
        pp = lax.dot_general(onehot, hn, (((0,), (0,)), ((), ())),
                             preferred_element_type=jnp.float32,
                             precision=_PREC)

        @pl.when(i == 0)
        def _():
            pool_ref[...] = pp

        @pl.when(i > 0)
        def _():
            pool_ref[...] += pp


def _layer(h, agg, w1, b1, w2, b2, g, bb, batch2):
    return pl.pallas_call(
        _layer_body,
        grid=(2, _GRID),
        in_specs=[
            pl.BlockSpec((_RB, _H), lambda p, i: ((1 - p) * i, 0)),
            pl.BlockSpec((1, _RB, _H), lambda p, i: (0, (1 - p) * i, 0)),
            pl.BlockSpec((1, _RB, _H), lambda p, i: (1, (1 - p) * i, 0)),
            pl.BlockSpec((_H, _H), lambda p, i: (0, 0)),
            pl.BlockSpec((1, _H), lambda p, i: (0, 0)),
            pl.BlockSpec((_H, _H), lambda p, i: (0, 0)),
            pl.BlockSpec((1, _H), lambda p, i: (0, 0)),
            pl.BlockSpec((1, _H), lambda p, i: (0, 0)),
            pl.BlockSpec((1, _H), lambda p, i: (0, 0)),
            pl.BlockSpec((_RB, 1), lambda p, i: (p * i, 0)),
        ],
        out_specs=[
            pl.BlockSpec((_RB, _H), lambda p, i: (p * i, 0)),
            pl.BlockSpec((_G, _H), lambda p, i: (0, 0)),
        ],
        out_shape=[
            jax.ShapeDtypeStruct((_N, _H), jnp.float32),
            jax.ShapeDtypeStruct((_G, _H), jnp.float32),
        ],
        scratch_shapes=[
            pltpu.VMEM((_GRID, _RB, _H), jnp.float32),
            pltpu.VMEM((_GRID, _RB, _H), jnp.float32),
            pltpu.VMEM((1, _H), jnp.float32),
            pltpu.VMEM((1, _H), jnp.float32),
        ],
    )(h, agg, agg, w1, b1.reshape(1, _H), w2, b2.reshape(1, _H),
      g.reshape(1, _H), bb.reshape(1, _H), batch2)


def _mlp_body(h_ref, a0_ref, a1_ref, w1_ref, b1_ref, w2_ref, b2_ref,
              u_ref, s1_ref, s2_ref):
    i = pl.program_id(0)
    t = h_ref[...] + a0_ref[0] + a1_ref[0]
    t = jnp.maximum(lax.dot_general(t, w1_ref[...], _MM,
                                    preferred_element_type=jnp.float32,
                                    precision=_PREC) + b1_ref[...], 0.0)
    u = (lax.dot_general(t, w2_ref[...], _MM,
                         preferred_element_type=jnp.float32,
                         precision=_PREC) + b2_ref[...])
    u_ref[...] = u
    ps1 = jnp.sum(u, axis=0, keepdims=True)
    ps2 = jnp.sum(u * u, axis=0, keepdims=True)

    @pl.when(i == 0)
    def _():
        s1_ref[...] = ps1
        s2_ref[...] = ps2

    @pl.when(i > 0)
    def _():
        s1_ref[...] += ps1
        s2_ref[...] += ps2


def _mlp(h, agg, w1, b1, w2, b2):
    return pl.pallas_call(
        _mlp_body,
        grid=(_GRID,),
        in_specs=[
            pl.BlockSpec((_RB, _H), lambda i: (i, 0)),
            pl.BlockSpec((1, _RB, _H), lambda i: (0, i, 0)),
            pl.BlockSpec((1, _RB, _H), lambda i: (1, i, 0)),
            pl.BlockSpec((_H, _H), lambda i: (0, 0)),
            pl.BlockSpec((1, _H), lambda i: (0, 0)),
            pl.BlockSpec((_H, _H), lambda i: (0, 0)),
            pl.BlockSpec((1, _H), lambda i: (0, 0)),
        ],
        out_specs=[
            pl.BlockSpec((_RB, _H), lambda i: (i, 0)),
            pl.BlockSpec((1, _H), lambda i: (0, 0)),
            pl.BlockSpec((1, _H), lambda i: (0, 0)),
        ],
        out_shape=[
            jax.ShapeDtypeStruct((_N, _H), jnp.float32),
            jax.ShapeDtypeStruct((1, _H), jnp.float32),
            jax.ShapeDtypeStruct((1, _H), jnp.float32),
        ],
    )(h, agg, agg, w1, b1.reshape(1, _H), w2, b2.reshape(1, _H))


def _bnres_body(u_ref, s1_ref, s2_ref, g_ref, b_ref, h_ref, batch_ref,
                hn_ref, pool_ref):
    i = pl.program_id(0)
    m = s1_ref[...] / _N
    v = s2_ref[...] / _N - m * m
    inv = lax.rsqrt(v + 1e-5)
    t = (u_ref[...] - m) * inv * g_ref[...] + b_ref[...]
    hn = jnp.maximum(t, 0.0) + h_ref[...]
    hn_ref[...] = hn
    onehot = (batch_ref[...] ==
              lax.broadcasted_iota(jnp.int32, (_RB, _G), 1)).astype(jnp.float32)
    pp = lax.dot_general(onehot, hn, (((0,), (0,)), ((), ())),
                         preferred_element_type=jnp.float32, precision=_PREC)

    @pl.when(i == 0)
    def _():
        pool_ref[...] = pp

    @pl.when(i > 0)
    def _():
        pool_ref[...] += pp


def _bnres(u, s1, s2, g, b, h, batch2):
    return pl.pallas_call(
        _bnres_body,
        grid=(_GRID,),
        in_specs=[
            pl.BlockSpec((_RB, _H), lambda i: (i, 0)),
            pl.BlockSpec((1, _H), lambda i: (0, 0)),
            pl.BlockSpec((1, _H), lambda i: (0, 0)),
            pl.BlockSpec((1, _H), lambda i: (0, 0)),
            pl.BlockSpec((1, _H), lambda i: (0, 0)),
            pl.BlockSpec((_RB, _H), lambda i: (i, 0)),
            pl.BlockSpec((_RB, 1), lambda i: (i, 0)),
        ],
        out_specs=[
            pl.BlockSpec((_RB, _H), lambda i: (i, 0)),
            pl.BlockSpec((_G, _H), lambda i: (0, 0)),
        ],
        out_shape=[
            jax.ShapeDtypeStruct((_N, _H), jnp.float32),
            jax.ShapeDtypeStruct((_G, _H), jnp.float32),
        ],
    )(u, s1, s2, g.reshape(1, _H), b.reshape(1, _H), h, batch2)


def _final_body(p_ref, w_ref, b_ref, o_ref):
    acc = jnp.zeros((_G, _OUT), jnp.float32)
    for i in range(_NL + 1):
        acc = acc + lax.dot_general(p_ref[i], w_ref[i], _MM,
                                    preferred_element_type=jnp.float32,
                                    precision=_PREC)
    o_ref[...] = acc + jnp.sum(b_ref[...], axis=0, keepdims=True)


def _final(pools, w, b):
    return pl.pallas_call(
        _final_body,
        out_shape=jax.ShapeDtypeStruct((_G, _OUT), jnp.float32),
    )(pools, w, b)


def kernel(x, edge_index, cycle_index, batch, params):
    p = params

    def _nblk(num_edges):
        per_tile = -(-num_edges // _NW)
        nb = -(-per_tile // _BLK)
        return -(-nb // _ISLOT) * _ISLOT

    nblk_e = _nblk(edge_index.shape[1])
    nblk_c = _nblk(cycle_index.shape[1])
    agg_e = _make_agg(nblk_e)
    agg_c = _make_agg(nblk_c)
    esrc, edst = _prep_edges(edge_index, nblk_e, nblk_e)
    csrc, cdst = _prep_edges(cycle_index, nblk_c, nblk_c)
    zeros = jnp.zeros((_NPAD, _H), jnp.float32)
    batch2 = batch.astype(jnp.int32).reshape(_N, 1)

    x0 = _emb(x, p["emb_w"], p["emb_b"])

    cagg = agg_c(x0, csrc, cdst, zeros)

    pools = []
    h = x0
    for i in range(_NL):
        eagg = agg_e(h, esrc, edst, zeros)
        h, pool = _layer(h, eagg, p["conv_w1"][i], p["conv_b1"][i],
                         p["conv_w2"][i], p["conv_b2"][i],
                         p["bn_g"][i], p["bn_b"][i], batch2)
        pools.append(pool)

    h4, pool4 = _layer(x0, cagg, p["conv2_w1"], p["conv2_b1"],
                       p["conv2_w2"], p["conv2_b2"],
                       p["bn2_g"], p["bn2_b"], batch2)
    pools.append(pool4)

    return _final(jnp.stack(pools), p["lin_w"], p["lin_b"])


# speedup vs baseline: 5.0561x; 1.0298x over previous
import functools

import jax
import jax.numpy as jnp
from jax import lax
from jax.experimental import pallas as pl
from jax.experimental.pallas import tpu as pltpu
from jax.experimental.pallas import tpu_sc as plsc

_N = 10000
_H = 128
_G = 64
_OUT = 64
_NL = 3

_NC = 2
_NS = 16
_NW = _NC * _NS
_BLK = 128
_NPAD = _N + 112
_RPT = _NPAD // _NS

_RB = 2000
_GRID = _N // _RB

_MM = (((1,), (0,)), ((), ()))
_PREC = None


_NBUF = 2
_ISLOT = 2 * _NBUF
_NBLK_PAD = 80


def _make_agg(nblk):
    mesh = plsc.VectorSubcoreMesh(core_axis_name="c", subcore_axis_name="s",
                                  num_cores=_NC, num_subcores=_NS)

    @functools.partial(
        pl.kernel,
        mesh=mesh,
        out_type=jax.ShapeDtypeStruct((_NC, _NPAD, _H), jnp.float32),
        scratch_types=[
            pltpu.VMEM((_ISLOT, _BLK), jnp.int32),
            pltpu.VMEM((_ISLOT, _BLK), jnp.int32),
            pltpu.VMEM((_NBUF, _BLK, _H), jnp.float32),
            pltpu.VMEM_SHARED((_NPAD, _H), jnp.float32),
        ] + [pltpu.SemaphoreType.DMA] * (2 * _NBUF + _ISLOT + 1),
    )
    def agg(h_hbm, src_hbm, dst_hbm, zeros_hbm, out_hbm,
            src_r, dst_r, rows_v, acc_sh, *sems):
        gsems = sems[:_NBUF]
        ssems = sems[_NBUF:2 * _NBUF]
        isems = sems[2 * _NBUF:2 * _NBUF + _ISLOT]
        zsem = sems[-1]
        c = lax.axis_index("c")
        s = lax.axis_index("s")
        wid = c * _NS + s
        zdesc = pltpu.async_copy(zeros_hbm.at[pl.ds(s * _RPT, _RPT)],
                                 acc_sh.at[pl.ds(s * _RPT, _RPT)], zsem)

        def fetch_idx(j, slot):
            pltpu.async_copy(src_hbm.at[wid].at[j], src_r.at[slot],
                             isems[slot])
            pltpu.async_copy(dst_hbm.at[wid].at[j], dst_r.at[slot],
                             isems[slot])

        def wait_idx(j, slot):
            pltpu.make_async_copy(src_hbm.at[wid].at[j], src_r.at[slot],
                                  isems[slot]).wait()
            pltpu.make_async_copy(dst_hbm.at[wid].at[j], dst_r.at[slot],
                                  isems[slot]).wait()

        def fetch_rows(slot, b):
            pltpu.async_copy(h_hbm.at[src_r.at[slot]], rows_v.at[b], gsems[b])

        def wait_rows(slot, b):
            pltpu.make_async_copy(h_hbm.at[src_r.at[slot]], rows_v.at[b],
                                  gsems[b]).wait()

        def scatter_rows(slot, b):
            pltpu.async_copy(rows_v.at[b], acc_sh.at[dst_r.at[slot]],
                             ssems[b], add=True)

        def wait_scatter(slot, b):
            pltpu.make_async_copy(rows_v.at[b], acc_sh.at[dst_r.at[slot]],
                                  ssems[b]).wait()

        for t in range(_ISLOT):
            fetch_idx(t, t)
        for b in range(_NBUF):
            wait_idx(b, b)
            fetch_rows(b, b)
        zdesc.wait()
        plsc.subcore_barrier()

        def round_body(r, carry):
            j0 = r * _ISLOT
            for t in range(_ISLOT):
                j = j0 + t
                slot = t
                b = t % _NBUF
                jn = j + _NBUF
                sn = (t + _NBUF) % _ISLOT
                wait_rows(slot, b)
                scatter_rows(slot, b)

                @pl.when(jn < nblk)
                def _():
                    wait_idx(jn, sn)

                wait_scatter(slot, b)

                @pl.when(j + _ISLOT < nblk)
                def _():
                    fetch_idx(j + _ISLOT, slot)

                @pl.when(jn < nblk)
                def _():
                    fetch_rows(sn, b)
            return carry

        lax.fori_loop(0, nblk // _ISLOT, round_body, 0)
        plsc.subcore_barrier()
        pltpu.sync_copy(acc_sh.at[pl.ds(s * _RPT, _RPT)],
                        out_hbm.at[c].at[pl.ds(s * _RPT, _RPT)])

    return agg


def _prep_edges(idx2, nblk, nblk_pad):
    total = _NW * nblk * _BLK
    pad = total - idx2.shape[1]
    pad_i = jnp.arange(pad, dtype=jnp.int32)
    src = jnp.concatenate([idx2[0].astype(jnp.int32), (pad_i * 97) % _N])
    dst = jnp.concatenate([idx2[1].astype(jnp.int32),
                           _N + (pad_i % (_NPAD - _N))])
    src = src.reshape(_NW, nblk, _BLK)
    dst = dst.reshape(_NW, nblk, _BLK)
    bp = nblk_pad - nblk
    if bp:
        blk_i = jnp.arange(_NW * bp * _BLK, dtype=jnp.int32)
        src = jnp.concatenate(
            [src, ((blk_i * 89) % _N).reshape(_NW, bp, _BLK)], axis=1)
        dst = jnp.concatenate(
            [dst, (_N + (blk_i % (_NPAD - _N))).reshape(_NW, bp, _BLK)],
            axis=1)
    return src, dst


def _emb_body(x_ref, w_ref, b_ref, o_ref):
    o_ref[...] = (lax.dot_general(x_ref[...], w_ref[...], _MM,
                                  preferred_element_type=jnp.float32,
                                  precision=_PREC) + b_ref[...])


def _emb(x, w, b):
    return pl.pallas_call(
        _emb_body,
        grid=(_GRID,),
        in_specs=[
            pl.BlockSpec((_RB, _H), lambda i: (i, 0)),
            pl.BlockSpec((_H, _H), lambda i: (0, 0)),
            pl.BlockSpec((1, _H), lambda i: (0, 0)),
        ],
        out_specs=pl.BlockSpec((_RB, _H), lambda i: (i, 0)),
        out_shape=jax.ShapeDtypeStruct((_N, _H), jnp.float32),
    )(x, w, b.reshape(1, _H))


def _layer_body(h_ref, a0_ref, a1_ref, w1_ref, b1_ref, w2_ref, b2_ref,
                g_ref, bb_ref, batch_ref, hn_ref, pool_ref,
                u_s, h_s, s1_s, s2_s):
    p = pl.program_id(0)
    i = pl.program_id(1)

    @pl.when(p == 0)
    def _():
        hb = h_ref[...]
        t = hb + a0_ref[0] + a1_ref[0]
        t = jnp.maximum(lax.dot_general(t, w1_ref[...], _MM,
                                        preferred_element_type=jnp.float32,
                                        precision=_PREC) + b1_ref[...], 0.0)
        u = (lax.dot_general(t, w2_ref[...], _MM,
                             preferred_element_type=jnp.float32,
                             precision=_PREC) + b2_ref[...])
        u_s[i] = u
        h_s[i] = hb
        ps1 = jnp.sum(u, axis=0, keepdims=True)
        ps2 = jnp.sum(u * u, axis=0, keepdims=True)

        @pl.when(i == 0)
        def _():
            s1_s[...] = ps1
            s2_s[...] = ps2

        @pl.when(i > 0)
        def _():
            s1_s[...] += ps1
            s2_s[...] += ps2

    @pl.when(p == 1)
    def _():
        m = s1_s[...] / _N
        v = s2_s[...] / _N - m * m
        inv = lax.rsqrt(v + 1e-5)
        t = (u_s[i] - m) * inv * g_ref[...] + bb_ref[...]
        hn = jnp.maximum(t, 0.0) + h_s[i]
        hn_ref[...] = hn
        onehot = (batch_ref[...] ==
                  lax.broadcasted_iota(jnp.int32, (_RB, _G), 1)
                  ).astype(jnp.float32)
        pp = lax.dot_general(onehot, hn, (((0,), (0,)), ((), ())),
                             preferred_element_type=jnp.float32,
                             precision=_PREC)

        @pl.when(i == 0)
        def _():
            pool_ref[...] = pp

        @pl.when(i > 0)
        def _():
            pool_ref[...] += pp


def _layer(h, agg, w1, b1, w2, b2, g, bb, batch2):
    return pl.pallas_call(
        _layer_body,
        grid=(2, _GRID),
        in_specs=[
            pl.BlockSpec((_RB, _H), lambda p, i: ((1 - p) * i, 0)),
            pl.BlockSpec((1, _RB, _H), lambda p, i: (0, (1 - p) * i, 0)),
            pl.BlockSpec((1, _RB, _H), lambda p, i: (1, (1 - p) * i, 0)),
            pl.BlockSpec((_H, _H), lambda p, i: (0, 0)),
            pl.BlockSpec((1, _H), lambda p, i: (0, 0)),
            pl.BlockSpec((_H, _H), lambda p, i: (0, 0)),
            pl.BlockSpec((1, _H), lambda p, i: (0, 0)),
            pl.BlockSpec((1, _H), lambda p, i: (0, 0)),
            pl.BlockSpec((1, _H), lambda p, i: (0, 0)),
            pl.BlockSpec((_RB, 1), lambda p, i: (p * i, 0)),
        ],
        out_specs=[
            pl.BlockSpec((_RB, _H), lambda p, i: (p * i, 0)),
            pl.BlockSpec((_G, _H), lambda p, i: (0, 0)),
        ],
        out_shape=[
            jax.ShapeDtypeStruct((_N, _H), jnp.float32),
            jax.ShapeDtypeStruct((_G, _H), jnp.float32),
        ],
        scratch_shapes=[
            pltpu.VMEM((_GRID, _RB, _H), jnp.float32),
            pltpu.VMEM((_GRID, _RB, _H), jnp.float32),
            pltpu.VMEM((1, _H), jnp.float32),
            pltpu.VMEM((1, _H), jnp.float32),
        ],
    )(h, agg, agg, w1, b1.reshape(1, _H), w2, b2.reshape(1, _H),
      g.reshape(1, _H), bb.reshape(1, _H), batch2)


def _mlp_body(h_ref, a0_ref, a1_ref, w1_ref, b1_ref, w2_ref, b2_ref,
              u_ref, s1_ref, s2_ref):
    i = pl.program_id(0)
    t = h_ref[...] + a0_ref[0] + a1_ref[0]
    t = jnp.maximum(lax.dot_general(t, w1_ref[...], _MM,
                                    preferred_element_type=jnp.float32,
                                    precision=_PREC) + b1_ref[...], 0.0)
    u = (lax.dot_general(t, w2_ref[...], _MM,
                         preferred_element_type=jnp.float32,
                         precision=_PREC) + b2_ref[...])
    u_ref[...] = u
    ps1 = jnp.sum(u, axis=0, keepdims=True)
    ps2 = jnp.sum(u * u, axis=0, keepdims=True)

    @pl.when(i == 0)
    def _():
        s1_ref[...] = ps1
        s2_ref[...] = ps2

    @pl.when(i > 0)
    def _():
        s1_ref[...] += ps1
        s2_ref[...] += ps2


def _mlp(h, agg, w1, b1, w2, b2):
    return pl.pallas_call(
        _mlp_body,
        grid=(_GRID,),
        in_specs=[
            pl.BlockSpec((_RB, _H), lambda i: (i, 0)),
            pl.BlockSpec((1, _RB, _H), lambda i: (0, i, 0)),
            pl.BlockSpec((1, _RB, _H), lambda i: (1, i, 0)),
            pl.BlockSpec((_H, _H), lambda i: (0, 0)),
            pl.BlockSpec((1, _H), lambda i: (0, 0)),
            pl.BlockSpec((_H, _H), lambda i: (0, 0)),
            pl.BlockSpec((1, _H), lambda i: (0, 0)),
        ],
        out_specs=[
            pl.BlockSpec((_RB, _H), lambda i: (i, 0)),
            pl.BlockSpec((1, _H), lambda i: (0, 0)),
            pl.BlockSpec((1, _H), lambda i: (0, 0)),
        ],
        out_shape=[
            jax.ShapeDtypeStruct((_N, _H), jnp.float32),
            jax.ShapeDtypeStruct((1, _H), jnp.float32),
            jax.ShapeDtypeStruct((1, _H), jnp.float32),
        ],
    )(h, agg, agg, w1, b1.reshape(1, _H), w2, b2.reshape(1, _H))


def _bnres_body(u_ref, s1_ref, s2_ref, g_ref, b_ref, h_ref, batch_ref,
                hn_ref, pool_ref):
    i = pl.program_id(0)
    m = s1_ref[...] / _N
    v = s2_ref[...] / _N - m * m
    inv = lax.rsqrt(v + 1e-5)
    t = (u_ref[...] - m) * inv * g_ref[...] + b_ref[...]
    hn = jnp.maximum(t, 0.0) + h_ref[...]
    hn_ref[...] = hn
    onehot = (batch_ref[...] ==
              lax.broadcasted_iota(jnp.int32, (_RB, _G), 1)).astype(jnp.float32)
    pp = lax.dot_general(onehot, hn, (((0,), (0,)), ((), ())),
                         preferred_element_type=jnp.float32, precision=_PREC)

    @pl.when(i == 0)
    def _():
        pool_ref[...] = pp

    @pl.when(i > 0)
    def _():
        pool_ref[...] += pp


def _bnres(u, s1, s2, g, b, h, batch2):
    return pl.pallas_call(
        _bnres_body,
        grid=(_GRID,),
        in_specs=[
            pl.BlockSpec((_RB, _H), lambda i: (i, 0)),
            pl.BlockSpec((1, _H), lambda i: (0, 0)),
            pl.BlockSpec((1, _H), lambda i: (0, 0)),
            pl.BlockSpec((1, _H), lambda i: (0, 0)),
            pl.BlockSpec((1, _H), lambda i: (0, 0)),
            pl.BlockSpec((_RB, _H), lambda i: (i, 0)),
            pl.BlockSpec((_RB, 1), lambda i: (i, 0)),
        ],
        out_specs=[
            pl.BlockSpec((_RB, _H), lambda i: (i, 0)),
            pl.BlockSpec((_G, _H), lambda i: (0, 0)),
        ],
        out_shape=[
            jax.ShapeDtypeStruct((_N, _H), jnp.float32),
            jax.ShapeDtypeStruct((_G, _H), jnp.float32),
        ],
    )(u, s1, s2, g.reshape(1, _H), b.reshape(1, _H), h, batch2)


def _final_body(p_ref, w_ref, b_ref, o_ref):
    acc = jnp.zeros((_G, _OUT), jnp.float32)
    for i in range(_NL + 1):
        acc = acc + lax.dot_general(p_ref[i], w_ref[i], _MM,
                                    preferred_element_type=jnp.float32,
                                    precision=_PREC)
    o_ref[...] = acc + jnp.sum(b_ref[...], axis=0, keepdims=True)


def _final(pools, w, b):
    return pl.pallas_call(
        _final_body,
        out_shape=jax.ShapeDtypeStruct((_G, _OUT), jnp.float32),
    )(pools, w, b)


def kernel(x, edge_index, cycle_index, batch, params):
    p = params

    def _nblk(num_edges):
        per_tile = -(-num_edges // _NW)
        nb = -(-per_tile // _BLK)
        return -(-nb // _ISLOT) * _ISLOT

    nblk_e = _nblk(edge_index.shape[1])
    nblk_c = _nblk(cycle_index.shape[1])
    agg_e = _make_agg(nblk_e)
    agg_c = _make_agg(nblk_c)
    esrc, edst = _prep_edges(edge_index, nblk_e, nblk_e)
    csrc, cdst = _prep_edges(cycle_index, nblk_c, nblk_c)
    zeros = jnp.zeros((_NPAD, _H), jnp.float32)
    batch2 = batch.astype(jnp.int32).reshape(_N, 1)

    x0 = _emb(x, p["emb_w"], p["emb_b"])

    cagg = agg_c(x0, csrc, cdst, zeros)

    pools = []
    h = x0
    for i in range(_NL):
        eagg = agg_e(h, esrc, edst, zeros)
        h, pool = _layer(h, eagg, p["conv_w1"][i], p["conv_b1"][i],
                         p["conv_w2"][i], p["conv_b2"][i],
                         p["bn_g"][i], p["bn_b"][i], batch2)
        pools.append(pool)

    h4, pool4 = _layer(x0, cagg, p["conv2_w1"], p["conv2_b1"],
                       p["conv2_w2"], p["conv2_b2"],
                       p["bn2_g"], p["bn2_b"], batch2)
    pools.append(pool4)

    return _final(jnp.stack(pools), p["lin_w"], p["lin_b"])
